# baseline (speedup 1.0000x reference)
import functools

import jax
import jax.numpy as jnp
from jax import lax
from jax.experimental import pallas as pl
from jax.experimental.pallas import tpu as pltpu
from jax.experimental.pallas import tpu_sc as plsc

B = 16384
HB = B // 2
D = 99
N_DATASETS = 100000

NC, NS = 2, 16
NW = NC * NS
ROUNDS = (D + NW - 1) // NW


def _build_sc_call():
    mesh = plsc.VectorSubcoreMesh(
        core_axis_name="c", subcore_axis_name="s",
        num_cores=NC, num_subcores=NS)

    @functools.partial(
        pl.kernel,
        mesh=mesh,
        compiler_params=pltpu.CompilerParams(needs_layout_passes=False),
        out_type=jax.ShapeDtypeStruct((D, B), jnp.float32),
        scratch_types=[
            pltpu.VMEM((N_DATASETS,), jnp.float32),
            pltpu.VMEM((HB,), jnp.int32),
            pltpu.VMEM((B,), jnp.float32),
            pltpu.VMEM((D + 16,), jnp.float32),
            pltpu.VMEM((D + 16,), jnp.float32),
        ],
    )
    def sc_call(u_hbm, lap_hbm, idx_hbm, at_hbm, alt_hbm, out_hbm,
                rowb, idb, acc, u_v, lap_v):
        wid = lax.axis_index("s") * NC + lax.axis_index("c")
        pltpu.sync_copy(u_hbm, u_v.at[pl.ds(0, D)])
        pltpu.sync_copy(lap_hbm, lap_v.at[pl.ds(0, D)])

        for rnd in range(ROUNDS):
            c = wid + rnd * NW

            @pl.when(c < D)
            def _():
                u_c = u_v[pl.ds(c, 16)][0]
                lap_c = lap_v[pl.ds(c, 16)][0]
                pltpu.sync_copy(at_hbm.at[c], rowb)

                def gather_mul(h, coef, add):
                    pltpu.sync_copy(idx_hbm.at[pl.ds(h * HB, HB)], idb)

                    def body(v, carry):
                        idvec = idb[pl.ds(v * 16, 16)]
                        val = plsc.load_gather(rowb, [idvec]) * coef
                        if add:
                            val = val + acc[pl.ds(h * HB + v * 16, 16)]
                        acc[pl.ds(h * HB + v * 16, 16)] = val
                        return carry
                    lax.fori_loop(0, HB // 16, body, 0)

                gather_mul(0, u_c, False)
                gather_mul(1, u_c, False)
                pltpu.sync_copy(alt_hbm.at[c], rowb)
                gather_mul(0, lap_c, True)
                gather_mul(1, lap_c, True)
                pltpu.sync_copy(acc, out_hbm.at[c])

    return sc_call


_SC_CALL = None


def kernel(x, data_id, frame, a_, b_, alpha_):
    global _SC_CALL
    if _SC_CALL is None:
        _SC_CALL = _build_sc_call()
    u = x[:, 0]
    lap = x[:, 2]
    idx = data_id.astype(jnp.int32)
    out_t = _SC_CALL(u, lap, idx, a_.T, alpha_.T)
    return out_t.T


# speedup vs baseline: 1.2011x; 1.2011x over previous
import functools

import jax
import jax.numpy as jnp
from jax import lax
from jax.experimental import pallas as pl
from jax.experimental.pallas import tpu as pltpu
from jax.experimental.pallas import tpu_sc as plsc

B = 16384
HB = B // 2
D = 99
N_DATASETS = 100000

NC, NS = 2, 16
NW = NC * NS
ROUNDS = (D + NW - 1) // NW


def _build_sc_call():
    mesh = plsc.VectorSubcoreMesh(
        core_axis_name="c", subcore_axis_name="s",
        num_cores=NC, num_subcores=NS)

    @functools.partial(
        pl.kernel,
        mesh=mesh,
        compiler_params=pltpu.CompilerParams(needs_layout_passes=False),
        out_type=jax.ShapeDtypeStruct((D, B), jnp.float32),
        scratch_types=[
            pltpu.VMEM((N_DATASETS,), jnp.float32),
            pltpu.VMEM((HB,), jnp.int32),
            pltpu.VMEM((B,), jnp.float32),
            pltpu.VMEM((D + 16,), jnp.float32),
            pltpu.VMEM((D + 16,), jnp.float32),
        ],
    )
    def sc_call(u_hbm, lap_hbm, idx_hbm, at_hbm, alt_hbm, out_hbm,
                rowb, idb, acc, u_v, lap_v):
        wid = lax.axis_index("s") * NC + lax.axis_index("c")
        pltpu.sync_copy(u_hbm, u_v.at[pl.ds(0, D)])
        pltpu.sync_copy(lap_hbm, lap_v.at[pl.ds(0, D)])

        UNROLL = 8

        def gather_mul(i0, n, coef, add):
            def body(v, carry):
                w0 = v * (16 * UNROLL)
                for s in range(UNROLL):
                    o = w0 + s * 16
                    idvec = idb[pl.ds(o, 16)]
                    val = plsc.load_gather(rowb, [idvec]) * coef
                    if add:
                        val = val + acc[pl.ds(i0 + o, 16)]
                    acc[pl.ds(i0 + o, 16)] = val
                return carry
            lax.fori_loop(0, n // (16 * UNROLL), body, 0)

        def do_feature(c, ranges):
            u_c = u_v[pl.ds(c, 16)][0]
            lap_c = lap_v[pl.ds(c, 16)][0]
            pltpu.sync_copy(at_hbm.at[c], rowb)
            for i0, n in ranges:
                pltpu.sync_copy(idx_hbm.at[pl.ds(i0, n)], idb.at[pl.ds(0, n)])
                gather_mul(i0, n, u_c, False)
            pltpu.sync_copy(alt_hbm.at[c], rowb)
            for i0, n in ranges:
                pltpu.sync_copy(idx_hbm.at[pl.ds(i0, n)], idb.at[pl.ds(0, n)])
                gather_mul(i0, n, lap_c, True)
            for i0, n in ranges:
                pltpu.sync_copy(acc.at[pl.ds(i0, n)],
                                out_hbm.at[c, pl.ds(i0, n)])

        for rnd in range(3):
            do_feature(wid + rnd * NW, [(0, HB), (HB, HB)])

        TAIL = D - 3 * NW
        QB = B // 4

        @pl.when(wid < 4 * TAIL)
        def _():
            c = 3 * NW + wid // 4
            q = wid % 4
            do_feature(c, [(q * QB, QB)])

    return sc_call


_SC_CALL = None


def kernel(x, data_id, frame, a_, b_, alpha_):
    global _SC_CALL
    if _SC_CALL is None:
        _SC_CALL = _build_sc_call()
    u = x[:, 0]
    lap = x[:, 2]
    idx = data_id.astype(jnp.int32)
    out_t = _SC_CALL(u, lap, idx, a_.T, alpha_.T)
    return out_t.T
